# Initial kernel scaffold; baseline (speedup 1.0000x reference)
import jax
import jax.numpy as jnp
from jax.experimental import pallas as pl


def kernel(q, k, v, indices):
    raise NotImplementedError("write your pallas kernel here")


# speedup vs baseline: 1.0207x; 1.0207x over previous
import jax
import jax.numpy as jnp
from jax.experimental import pallas as pl
from jax.experimental.pallas import tpu as pltpu

SCALE_ = 0.125
WS_ = 14
W2_ = WS_ * WS_
NWS_ = 16
NW_ = NWS_ * NWS_


def _attn_body(idx_ref, q_ref, k0_ref, k1_ref, v0_ref, v1_ref, o_ref):
    q = q_ref[0, 0]
    k0 = k0_ref[0, 0]
    k1 = k1_ref[0, 0]
    v0 = v0_ref[0, 0]
    v1 = v1_ref[0, 0]
    s0 = jax.lax.dot_general(q, k0, (((1,), (1,)), ((), ())),
                             preferred_element_type=jnp.float32) * SCALE_
    s1 = jax.lax.dot_general(q, k1, (((1,), (1,)), ((), ())),
                             preferred_element_type=jnp.float32) * SCALE_
    m = jnp.maximum(jnp.max(s0, axis=-1, keepdims=True),
                    jnp.max(s1, axis=-1, keepdims=True))
    p0 = jnp.exp(s0 - m)
    p1 = jnp.exp(s1 - m)
    l = (jnp.sum(p0, axis=-1, keepdims=True)
         + jnp.sum(p1, axis=-1, keepdims=True))
    o = (jax.lax.dot_general(p0, v0, (((1,), (0,)), ((), ())),
                             preferred_element_type=jnp.float32)
         + jax.lax.dot_general(p1, v1, (((1,), (0,)), ((), ())),
                               preferred_element_type=jnp.float32))
    o_ref[0, 0] = o / l


def kernel(q, k, v, indices):
    B, heads, H, W, C = q.shape
    nws = H // WS_
    nw = nws * nws
    w2 = W2_

    def to_windows(x):
        x = x.reshape(heads, nws, WS_, nws, WS_, C)
        return x.transpose(0, 1, 3, 2, 4, 5).reshape(heads, nw, w2, C)

    qw = to_windows(q)
    kw = to_windows(k)
    vw = to_windows(v)
    idx = indices.reshape(heads, nw, -1).astype(jnp.int32)

    def qmap(h, w, idx_ref):
        return (h, w, 0, 0)

    def gmap(t):
        def m(h, w, idx_ref):
            return (h, idx_ref[h, w, t], 0, 0)
        return m

    blk = pl.BlockSpec((1, 1, w2, C), qmap)
    grid_spec = pltpu.PrefetchScalarGridSpec(
        num_scalar_prefetch=1,
        grid=(heads, nw),
        in_specs=[
            blk,
            pl.BlockSpec((1, 1, w2, C), gmap(0)),
            pl.BlockSpec((1, 1, w2, C), gmap(1)),
            pl.BlockSpec((1, 1, w2, C), gmap(0)),
            pl.BlockSpec((1, 1, w2, C), gmap(1)),
        ],
        out_specs=blk,
    )
    out = pl.pallas_call(
        _attn_body,
        grid_spec=grid_spec,
        out_shape=jax.ShapeDtypeStruct((heads, nw, w2, C), jnp.float32),
    )(idx, qw, kw, kw, vw, vw)

    x = out.reshape(heads, nws, nws, WS_, WS_, C)
    x = x.transpose(1, 3, 2, 4, 0, 5).reshape(B, H, W, heads * C)
    return x


# speedup vs baseline: 1.1465x; 1.1233x over previous
import jax
import jax.numpy as jnp
from jax.experimental import pallas as pl
from jax.experimental.pallas import tpu as pltpu

SCALE_ = 0.125
WS_ = 14
W2_ = WS_ * WS_
NWS_ = 16
NW_ = NWS_ * NWS_


def _attn_body(idx_ref, q_ref, k0_ref, k1_ref, v0_ref, v1_ref, o_ref):
    h = pl.program_id(1)
    q = q_ref[0, 0, :, 0, :, :].reshape(W2_, 64).astype(jnp.bfloat16)
    k0 = k0_ref[0, 0, :, 0, :, :].reshape(W2_, 64).astype(jnp.bfloat16)
    k1 = k1_ref[0, 0, :, 0, :, :].reshape(W2_, 64).astype(jnp.bfloat16)
    v0 = v0_ref[0, 0, :, 0, :, :].reshape(W2_, 64).astype(jnp.bfloat16)
    v1 = v1_ref[0, 0, :, 0, :, :].reshape(W2_, 64).astype(jnp.bfloat16)
    s0 = jax.lax.dot_general(q, k0, (((1,), (1,)), ((), ())),
                             preferred_element_type=jnp.float32) * SCALE_
    s1 = jax.lax.dot_general(q, k1, (((1,), (1,)), ((), ())),
                             preferred_element_type=jnp.float32) * SCALE_
    m = jnp.maximum(jnp.max(s0, axis=-1, keepdims=True),
                    jnp.max(s1, axis=-1, keepdims=True))
    p0 = jnp.exp(s0 - m)
    p1 = jnp.exp(s1 - m)
    l = (jnp.sum(p0, axis=-1, keepdims=True)
         + jnp.sum(p1, axis=-1, keepdims=True))
    o = (jax.lax.dot_general(p0.astype(jnp.bfloat16), v0,
                             (((1,), (0,)), ((), ())),
                             preferred_element_type=jnp.float32)
         + jax.lax.dot_general(p1.astype(jnp.bfloat16), v1,
                               (((1,), (0,)), ((), ())),
                               preferred_element_type=jnp.float32))
    o = (o / l).reshape(WS_, WS_, 64)
    o_ref[0, :, 0, :, h, :] = o


def kernel(q, k, v, indices):
    B, heads, H, W, C = q.shape
    nws = H // WS_
    nw = nws * nws

    qv = q.reshape(heads, nws, WS_, nws, WS_, C)
    kv = k.reshape(heads, nws, WS_, nws, WS_, C)
    vv = v.reshape(heads, nws, WS_, nws, WS_, C)
    idx = indices.reshape(heads, nw, -1).astype(jnp.int32)

    def qmap(w, h, idx_ref):
        return (h, w // nws, 0, w % nws, 0, 0)

    def gmap(t):
        def m(w, h, idx_ref):
            g = idx_ref[h, w, t]
            return (h, g // nws, 0, g % nws, 0, 0)
        return m

    def omap(w, h, idx_ref):
        return (w // nws, 0, w % nws, 0, 0, 0)

    in_blk = (1, 1, WS_, 1, WS_, C)
    grid_spec = pltpu.PrefetchScalarGridSpec(
        num_scalar_prefetch=1,
        grid=(nw, heads),
        in_specs=[
            pl.BlockSpec(in_blk, qmap),
            pl.BlockSpec(in_blk, gmap(0)),
            pl.BlockSpec(in_blk, gmap(1)),
            pl.BlockSpec(in_blk, gmap(0)),
            pl.BlockSpec(in_blk, gmap(1)),
        ],
        out_specs=pl.BlockSpec((1, WS_, 1, WS_, heads, C), omap),
    )
    out = pl.pallas_call(
        _attn_body,
        grid_spec=grid_spec,
        out_shape=jax.ShapeDtypeStruct((nws, WS_, nws, WS_, heads, C),
                                       jnp.float32),
    )(idx, qv, kv, kv, vv, vv)

    return out.reshape(B, H, W, heads * C)


# speedup vs baseline: 1.6163x; 1.4098x over previous
import jax
import jax.numpy as jnp
from jax.experimental import pallas as pl
from jax.experimental.pallas import tpu as pltpu

SCALE_ = 0.125
WS_ = 14
W2_ = WS_ * WS_
NWS_ = 16
HEADS_ = 4


def _attn_body(idx_ref, q_ref, *refs):
    k_refs = refs[0:8]
    v_refs = refs[8:16]
    o_ref = refs[16]
    for h in range(HEADS_):
        q = q_ref[h, 0, :, 0, :, :].reshape(W2_, 64).astype(jnp.bfloat16)
        k0 = k_refs[2 * h][0, 0, :, 0, :, :].reshape(W2_, 64).astype(jnp.bfloat16)
        k1 = k_refs[2 * h + 1][0, 0, :, 0, :, :].reshape(W2_, 64).astype(jnp.bfloat16)
        v0 = v_refs[2 * h][0, 0, :, 0, :, :].reshape(W2_, 64).astype(jnp.bfloat16)
        v1 = v_refs[2 * h + 1][0, 0, :, 0, :, :].reshape(W2_, 64).astype(jnp.bfloat16)
        s0 = jax.lax.dot_general(q, k0, (((1,), (1,)), ((), ())),
                                 preferred_element_type=jnp.float32) * SCALE_
        s1 = jax.lax.dot_general(q, k1, (((1,), (1,)), ((), ())),
                                 preferred_element_type=jnp.float32) * SCALE_
        m = jnp.maximum(jnp.max(s0, axis=-1, keepdims=True),
                        jnp.max(s1, axis=-1, keepdims=True))
        p0 = jnp.exp(s0 - m)
        p1 = jnp.exp(s1 - m)
        l = (jnp.sum(p0, axis=-1, keepdims=True)
             + jnp.sum(p1, axis=-1, keepdims=True))
        o = (jax.lax.dot_general(p0.astype(jnp.bfloat16), v0,
                                 (((1,), (0,)), ((), ())),
                                 preferred_element_type=jnp.float32)
             + jax.lax.dot_general(p1.astype(jnp.bfloat16), v1,
                                   (((1,), (0,)), ((), ())),
                                   preferred_element_type=jnp.float32))
        o_ref[0, :, 0, :, h, :] = (o / l).reshape(WS_, WS_, 64)


def kernel(q, k, v, indices):
    B, heads, H, W, C = q.shape
    nws = H // WS_
    nw = nws * nws

    qv = q.reshape(heads, nws, WS_, nws, WS_, C)
    kv = k.reshape(heads, nws, WS_, nws, WS_, C)
    vv = v.reshape(heads, nws, WS_, nws, WS_, C)
    idx = indices.reshape(heads, nw, -1).astype(jnp.int32)

    def qmap(w, idx_ref):
        return (0, w // nws, 0, w % nws, 0, 0)

    def gmap(h, t):
        def m(w, idx_ref):
            g = idx_ref[h, w, t]
            return (h, g // nws, 0, g % nws, 0, 0)
        return m

    def omap(w, idx_ref):
        return (w // nws, 0, w % nws, 0, 0, 0)

    g_blk = (1, 1, WS_, 1, WS_, C)
    gspecs = [pl.BlockSpec(g_blk, gmap(h, t))
              for h in range(heads) for t in range(2)]
    grid_spec = pltpu.PrefetchScalarGridSpec(
        num_scalar_prefetch=1,
        grid=(nw,),
        in_specs=[pl.BlockSpec((heads, 1, WS_, 1, WS_, C), qmap)]
                 + gspecs + gspecs,
        out_specs=pl.BlockSpec((1, WS_, 1, WS_, heads, C), omap),
    )
    out = pl.pallas_call(
        _attn_body,
        grid_spec=grid_spec,
        out_shape=jax.ShapeDtypeStruct((nws, WS_, nws, WS_, heads, C),
                                       jnp.float32),
    )(idx, qv, *([kv] * 8), *([vv] * 8))

    return out.reshape(B, H, W, heads * C)


# speedup vs baseline: 1.6204x; 1.0025x over previous
import jax
import jax.numpy as jnp
from jax.experimental import pallas as pl
from jax.experimental.pallas import tpu as pltpu

SCALE_ = 0.125
WS_ = 14
W2_ = WS_ * WS_
NWS_ = 16
HEADS_ = 4


def _fmt_body(x_ref, o_ref):
    xb = x_ref[0, 0].astype(jnp.bfloat16)
    for wj in range(NWS_):
        o_ref[0, 0, wj] = xb[:, WS_ * wj:WS_ * (wj + 1), :].reshape(W2_, 64)


def _attn_body(idx_ref, q_ref, *refs):
    k_refs = refs[0:8]
    v_refs = refs[8:16]
    o_ref = refs[16]
    for h in range(HEADS_):
        q = q_ref[h, 0, 0]
        k0 = k_refs[2 * h][0, 0, 0]
        k1 = k_refs[2 * h + 1][0, 0, 0]
        v0 = v_refs[2 * h][0, 0, 0]
        v1 = v_refs[2 * h + 1][0, 0, 0]
        s0 = jax.lax.dot_general(q, k0, (((1,), (1,)), ((), ())),
                                 preferred_element_type=jnp.float32) * SCALE_
        s1 = jax.lax.dot_general(q, k1, (((1,), (1,)), ((), ())),
                                 preferred_element_type=jnp.float32) * SCALE_
        m = jnp.maximum(jnp.max(s0, axis=-1, keepdims=True),
                        jnp.max(s1, axis=-1, keepdims=True))
        p0 = jnp.exp(s0 - m)
        p1 = jnp.exp(s1 - m)
        l = (jnp.sum(p0, axis=-1, keepdims=True)
             + jnp.sum(p1, axis=-1, keepdims=True))
        o = (jax.lax.dot_general(p0.astype(jnp.bfloat16), v0,
                                 (((1,), (0,)), ((), ())),
                                 preferred_element_type=jnp.float32)
             + jax.lax.dot_general(p1.astype(jnp.bfloat16), v1,
                                   (((1,), (0,)), ((), ())),
                                   preferred_element_type=jnp.float32))
        o_ref[h, 0, 0] = o / l


def _format(x, heads, nws, C):
    xv = x.reshape(heads, nws, WS_, nws * WS_, C)
    return pl.pallas_call(
        _fmt_body,
        grid=(heads, nws),
        in_specs=[pl.BlockSpec((1, 1, WS_, nws * WS_, C),
                               lambda h, wi: (h, wi, 0, 0, 0))],
        out_specs=pl.BlockSpec((1, 1, NWS_, W2_, C),
                               lambda h, wi: (h, wi, 0, 0, 0)),
        out_shape=jax.ShapeDtypeStruct((heads, nws, NWS_, W2_, C),
                                       jnp.bfloat16),
    )(xv)


def kernel(q, k, v, indices):
    B, heads, H, W, C = q.shape
    nws = H // WS_
    nw = nws * nws

    qf = _format(q, heads, nws, C)
    kf = _format(k, heads, nws, C)
    vf = _format(v, heads, nws, C)
    idx = indices.reshape(heads, nw, -1).astype(jnp.int32)

    def qmap(w, idx_ref):
        return (0, w // nws, w % nws, 0, 0)

    def gmap(h, t):
        def m(w, idx_ref):
            g = idx_ref[h, w, t]
            return (h, g // nws, g % nws, 0, 0)
        return m

    g_blk = (1, 1, 1, W2_, C)
    gspecs = [pl.BlockSpec(g_blk, gmap(h, t))
              for h in range(heads) for t in range(2)]
    grid_spec = pltpu.PrefetchScalarGridSpec(
        num_scalar_prefetch=1,
        grid=(nw,),
        in_specs=[pl.BlockSpec((heads, 1, 1, W2_, C), qmap)]
                 + gspecs + gspecs,
        out_specs=pl.BlockSpec((heads, 1, 1, W2_, C), qmap),
    )
    out = pl.pallas_call(
        _attn_body,
        grid_spec=grid_spec,
        out_shape=jax.ShapeDtypeStruct((heads, nws, nws, W2_, C),
                                       jnp.float32),
    )(idx, qf, *([kf] * 8), *([vf] * 8))

    x = out.reshape(heads, nws, nws, WS_, WS_, C)
    x = x.transpose(1, 3, 2, 4, 0, 5).reshape(B, H, W, heads * C)
    return x


# speedup vs baseline: 2.1235x; 1.3105x over previous
import jax
import jax.numpy as jnp
from jax.experimental import pallas as pl
from jax.experimental.pallas import tpu as pltpu

SCALE_ = 0.125
WS_ = 14
W2_ = WS_ * WS_
NWS_ = 16
HEADS_ = 4


def _fmt_body(x_ref, o_ref):
    xb = x_ref[0, 0].astype(jnp.bfloat16)
    xt = xb.transpose(0, 2, 1)
    for wj in range(NWS_):
        o_ref[0, 0, wj] = xt[:, WS_ * wj:WS_ * (wj + 1), :].reshape(W2_, 64)


def _attn_body(idx_ref, q_ref, *refs):
    k_refs = refs[0:8]
    v_refs = refs[8:16]
    o_ref = refs[16]
    for h in range(HEADS_):
        q = q_ref[h, 0, 0]
        k0 = k_refs[2 * h][0, 0, 0]
        k1 = k_refs[2 * h + 1][0, 0, 0]
        v0 = v_refs[2 * h][0, 0, 0]
        v1 = v_refs[2 * h + 1][0, 0, 0]
        s0 = jax.lax.dot_general(q, k0, (((1,), (1,)), ((), ())),
                                 preferred_element_type=jnp.float32) * SCALE_
        s1 = jax.lax.dot_general(q, k1, (((1,), (1,)), ((), ())),
                                 preferred_element_type=jnp.float32) * SCALE_
        m = jnp.maximum(jnp.max(s0, axis=-1, keepdims=True),
                        jnp.max(s1, axis=-1, keepdims=True))
        p0 = jnp.exp(s0 - m)
        p1 = jnp.exp(s1 - m)
        l = (jnp.sum(p0, axis=-1, keepdims=True)
             + jnp.sum(p1, axis=-1, keepdims=True))
        o = (jax.lax.dot_general(p0.astype(jnp.bfloat16), v0,
                                 (((1,), (0,)), ((), ())),
                                 preferred_element_type=jnp.float32)
             + jax.lax.dot_general(p1.astype(jnp.bfloat16), v1,
                                   (((1,), (0,)), ((), ())),
                                   preferred_element_type=jnp.float32))
        o_ref[0, :, 0, :, h, :] = (o / l).reshape(WS_, WS_, 64)


def _format(x, heads, nws, C):
    H = nws * WS_
    xt = x.transpose(0, 1, 2, 4, 3).reshape(heads, nws, WS_, C, H)
    return pl.pallas_call(
        _fmt_body,
        grid=(heads, nws),
        in_specs=[pl.BlockSpec((1, 1, WS_, C, H),
                               lambda h, wi: (h, wi, 0, 0, 0))],
        out_specs=pl.BlockSpec((1, 1, NWS_, W2_, C),
                               lambda h, wi: (h, wi, 0, 0, 0)),
        out_shape=jax.ShapeDtypeStruct((heads, nws, NWS_, W2_, C),
                                       jnp.bfloat16),
    )(xt)


def kernel(q, k, v, indices):
    B, heads, H, W, C = q.shape
    nws = H // WS_
    nw = nws * nws

    qf = _format(q, heads, nws, C)
    kf = _format(k, heads, nws, C)
    vf = _format(v, heads, nws, C)
    idx = indices.reshape(heads, nw, -1).astype(jnp.int32)

    def qmap(w, idx_ref):
        return (0, w // nws, w % nws, 0, 0)

    def gmap(h, t):
        def m(w, idx_ref):
            g = idx_ref[h, w, t]
            return (h, g // nws, g % nws, 0, 0)
        return m

    def omap(w, idx_ref):
        return (w // nws, 0, w % nws, 0, 0, 0)

    g_blk = (1, 1, 1, W2_, C)
    gspecs = [pl.BlockSpec(g_blk, gmap(h, t))
              for h in range(heads) for t in range(2)]
    grid_spec = pltpu.PrefetchScalarGridSpec(
        num_scalar_prefetch=1,
        grid=(nw,),
        in_specs=[pl.BlockSpec((heads, 1, 1, W2_, C), qmap)]
                 + gspecs + gspecs,
        out_specs=pl.BlockSpec((1, WS_, 1, WS_, heads, C), omap),
    )
    out = pl.pallas_call(
        _attn_body,
        grid_spec=grid_spec,
        out_shape=jax.ShapeDtypeStruct((nws, WS_, nws, WS_, heads, C),
                                       jnp.float32),
    )(idx, qf, *([kf] * 8), *([vf] * 8))

    return out.reshape(B, H, W, heads * C)
